# baseline (speedup 1.0000x reference)
import functools

import jax
import jax.numpy as jnp
import numpy as np
from jax import lax
from jax.experimental import pallas as pl
from jax.experimental.pallas import tpu as pltpu
from jax.experimental.pallas import tpu_sc as plsc

C = 384
K = 5
L = 16
NSLICE = C // L
BIG = np.int32(1 << 30)


def _sc_routing_body(bn_e_hbm, bn_n_hbm, out_hbm, ve_ref, vn_ref, out_ref,
                     tmp_ref):
    cid = lax.axis_index("c")
    sid = lax.axis_index("s")

    @pl.when(jnp.logical_and(cid == 0, sid == 0))
    def _leader():
        lane = lax.iota(jnp.int32, L)
        zeros = jnp.zeros((L,), jnp.int32)
        zerod = zeros + cid

        pltpu.sync_copy(bn_e_hbm, ve_ref)
        pltpu.sync_copy(bn_n_hbm, vn_ref)

        def splat_min_f32(x):
            s, _ = plsc.sort_key_val(x, zeros)
            tmp_ref[...] = plsc.bitcast(s, jnp.int32)
            return plsc.bitcast(plsc.load_gather(tmp_ref, [zerod]),
                                jnp.float32)

        def splat_min_i32(x):
            s, _ = plsc.sort_key_val(x, zeros)
            tmp_ref[...] = s
            return plsc.load_gather(tmp_ref, [zerod])

        def absify(vref):
            def body(i, _):
                vref[pl.ds(i * L, L)] = jnp.abs(vref[pl.ds(i * L, L)])
                return 0
            lax.fori_loop(0, NSLICE, body, 0)

        absify(ve_ref)
        absify(vn_ref)

        def top5_smallest(vref):
            idxvec = zeros
            for p in range(K):
                def scan_min(i, m):
                    return jnp.minimum(m, vref[pl.ds(i * L, L)])
                m = lax.fori_loop(0, NSLICE, scan_min,
                                  jnp.full((L,), jnp.inf, jnp.float32))
                mmin = splat_min_f32(m)

                def scan_arg(i, best):
                    v = vref[pl.ds(i * L, L)]
                    cand = jnp.where(v == mmin, lane + i * L, BIG)
                    return jnp.minimum(best, cand)
                best = lax.fori_loop(0, NSLICE, scan_arg,
                                     jnp.full((L,), BIG, jnp.int32))
                widx = splat_min_i32(best)

                def mask_out(i, _):
                    v = vref[pl.ds(i * L, L)]
                    vref[pl.ds(i * L, L)] = jnp.where(
                        lane + i * L == widx, jnp.inf, v)
                    return 0
                lax.fori_loop(0, NSLICE, mask_out, 0)
                idxvec = jnp.where(lane == p, widx, idxvec)
            return idxvec

        idx1 = top5_smallest(ve_ref)
        idx2 = top5_smallest(vn_ref)

        def write_route(pos_idx, src_idx, sel_row, src_row):
            key = jnp.where(lane < K, pos_idx, BIG)
            val = jnp.where(lane < K, src_idx, 0)
            k_s, v_s = plsc.sort_key_val(key, val)
            tmp_ref[...] = k_s
            a = [plsc.load_gather(tmp_ref, [zerod + k]) for k in range(K)]
            tmp_ref[...] = v_s
            b = [plsc.load_gather(tmp_ref, [zerod + k]) for k in range(K)]

            def body(i, _):
                c = lane + i * L
                sel = jnp.zeros((L,), jnp.int32)
                src = b[0]
                for k in range(K):
                    sel = jnp.where(c == a[k], 1, sel)
                    if k > 0:
                        src = jnp.where(a[k] <= c, b[k], src)
                out_ref[pl.ds(sel_row * C + i * L, L)] = sel
                out_ref[pl.ds(src_row * C + i * L, L)] = src
                return 0
            lax.fori_loop(0, NSLICE, body, 0)

        write_route(idx1, idx2, 0, 1)
        write_route(idx2, idx1, 2, 3)
        pltpu.sync_copy(out_ref, out_hbm)


@functools.partial(jax.jit, static_argnums=())
def _sc_routing(bn_e, bn_n):
    mesh = plsc.VectorSubcoreMesh(core_axis_name="c", subcore_axis_name="s")
    fn = pl.kernel(
        _sc_routing_body,
        out_type=jax.ShapeDtypeStruct((4 * C,), jnp.int32),
        mesh=mesh,
        scratch_types=[
            pltpu.VMEM((C,), jnp.float32),
            pltpu.VMEM((C,), jnp.float32),
            pltpu.VMEM((4 * C,), jnp.int32),
            pltpu.VMEM((L,), jnp.int32),
        ],
        compiler_params=pltpu.CompilerParams(needs_layout_passes=False),
    )
    return fn(bn_e, bn_n).reshape(4, C)


def _tc_copy_body(route_ref, e1, n1, n2, e2, x1, x2):
    c = pl.program_id(0)
    s1 = route_ref[0, c]
    s2 = route_ref[2, c]

    @pl.when(s1 == 0)
    def _():
        x1[...] = e1[...]

    @pl.when(s1 != 0)
    def _():
        x1[...] = n1[...]

    @pl.when(s2 == 0)
    def _():
        x2[...] = n2[...]

    @pl.when(s2 != 0)
    def _():
        x2[...] = e2[...]


def _tc_copy(route, e, n):
    blk = (e.shape[0], 1, e.shape[2], e.shape[3])
    grid_spec = pltpu.PrefetchScalarGridSpec(
        num_scalar_prefetch=1,
        grid=(C,),
        in_specs=[
            pl.BlockSpec(blk, lambda c, r: (0, c, 0, 0)),
            pl.BlockSpec(blk, lambda c, r: (0, r[1, c], 0, 0)),
            pl.BlockSpec(blk, lambda c, r: (0, c, 0, 0)),
            pl.BlockSpec(blk, lambda c, r: (0, r[3, c], 0, 0)),
        ],
        out_specs=[
            pl.BlockSpec(blk, lambda c, r: (0, c, 0, 0)),
            pl.BlockSpec(blk, lambda c, r: (0, c, 0, 0)),
        ],
    )
    return pl.pallas_call(
        _tc_copy_body,
        grid_spec=grid_spec,
        out_shape=[jax.ShapeDtypeStruct(e.shape, e.dtype)] * 2,
        compiler_params=pltpu.CompilerParams(
            dimension_semantics=("arbitrary",),
        ),
    )(route, e, n, n, e)


def kernel(e, n, bn_e, bn_n):
    route = _sc_routing(bn_e, bn_n)
    x1, x2 = _tc_copy(route, e, n)
    return (x1, x2)
